# baseline (speedup 1.0000x reference)
import jax
import jax.numpy as jnp
import numpy as np
from jax.experimental import pallas as pl
from jax.experimental.pallas import tpu as pltpu

_INDEX_ORDER = np.array([
    [0, 1, 5, 6, 14, 15, 27, 28],
    [2, 4, 7, 13, 16, 26, 29, 42],
    [3, 8, 12, 17, 25, 30, 41, 43],
    [9, 11, 18, 24, 31, 40, 44, 53],
    [10, 19, 23, 32, 39, 45, 52, 54],
    [20, 22, 33, 38, 46, 51, 55, 60],
    [21, 34, 37, 47, 50, 56, 59, 61],
    [35, 36, 48, 49, 57, 58, 62, 63]], dtype=np.int32).flatten()


def _permute_planes(x_hbm, o_hbm, sem):
    copies = []
    for j in range(64):
        a, d3 = divmod(int(_INDEX_ORDER[j]), 8)
        copies.append(pltpu.make_async_copy(
            x_hbm.at[:, a, :, d3, :],
            o_hbm.at[:, j // 8, :, j % 8, :],
            sem))
    for c in copies:
        c.start()
    for c in copies:
        c.wait()


def kernel(x):
    B, C = x.shape[0], x.shape[1]
    s = C // 128
    x5 = (x.transpose(0, 2, 3, 1)
           .reshape(B, 8, 8, s, 128)
           .transpose(0, 1, 3, 2, 4))

    y5 = pl.pallas_call(
        _permute_planes,
        in_specs=[pl.BlockSpec(memory_space=pltpu.MemorySpace.HBM)],
        out_specs=pl.BlockSpec(memory_space=pltpu.MemorySpace.HBM),
        out_shape=jax.ShapeDtypeStruct((B, 8, s, 8, 128), jnp.float32),
        scratch_shapes=[pltpu.SemaphoreType.DMA],
    )(x5)

    return (y5.transpose(0, 2, 4, 1, 3)
              .reshape(B, C, 64))
